# baseline (speedup 1.0000x reference)
import functools

import jax
import jax.numpy as jnp
from jax import lax
from jax.experimental import pallas as pl
from jax.experimental.pallas import tpu as pltpu
from jax.experimental.pallas import tpu_sc as plsc

N = 50000
D = 128
NPS = 12
ITER = 4
NREL = 15
CAT = NREL * D
RCAT = CAT - D
TAB_ROWS = N * RCAT // 32
ELR = 10000
E_TOT = NPS * N + 2 * ELR
GRP = 768
GPT = 52
NGRP = 16 * GPT
E_PAD = NGRP * GRP
RPT = 3128
ACC_ROWS = 16 * RPT
RBLK = 2000
GRID = N // RBLK

_HI = lax.Precision.HIGHEST


def _gn_tile(t, g, b):
    m = jnp.sum(t, axis=1, keepdims=True) * (1.0 / D)
    d = t - m
    v = jnp.sum(d * d, axis=1, keepdims=True) * (1.0 / D)
    return d * lax.rsqrt(v + 1e-5) * g + b


def _k1_body(c_ref, f_ref, w1, b1, w2, g1, bb1, ws1, bs1, ws2, gs, bbs, o_ref):
    def branch(inp, wa, ba, wb, g, b):
        x = inp[...]
        h = x[:, 0:1] * wa[0:1, :] + x[:, 1:2] * wa[1:2, :] + ba[...]
        h = jnp.maximum(h, 0.0)
        h = jnp.dot(h, wb[...], precision=_HI)
        return _gn_tile(h, g[...], b[...])

    x = branch(c_ref, w1, b1, w2, g1, bb1)
    y = branch(f_ref, ws1, bs1, ws2, gs, bbs)
    o_ref[...] = jnp.maximum(x + y, 0.0)


def _k2_body(f_ref, w_ref, ob_ref, om_ref):
    y = jnp.dot(f_ref[...], w_ref[...], precision=_HI)
    ob_ref[...] = y[:, :D]
    om_ref[...] = y[:, D:].astype(jnp.bfloat16)


def _k3_body(y_ref, s_ref, r_ref, gn_ref, bn_ref, w2_ref, g2_ref, b2_ref, o_ref):
    scat = jnp.concatenate(
        [s_ref[0], s_ref[1], s_ref[2], s_ref[3]], axis=1).astype(jnp.float32)
    temp = y_ref[...] + scat
    h = jnp.maximum(_gn_tile(temp, gn_ref[...], bn_ref[...]), 0.0)
    f2 = _gn_tile(jnp.dot(h, w2_ref[...], precision=_HI), g2_ref[...], b2_ref[...])
    o_ref[...] = jnp.maximum(f2 + r_ref[...], 0.0)


def _sc_scatter(tab, idxv, idxu, zeros, out,
                idxv_v, idxu_v, rows_v, accum,
                sem_g0, sem_g1, sem_i0, sem_i1):
    c = lax.axis_index("c")
    s = lax.axis_index("s")
    sem_g = (sem_g0, sem_g1)
    sem_i = (sem_i0, sem_i1)

    def fire_idx(q, g, b):
        pltpu.async_copy(idxv.at[q, g], idxv_v.at[b], sem_i[b])
        pltpu.async_copy(idxu.at[g], idxu_v.at[b], sem_i[b])

    def wait_idx(b):
        pltpu.make_async_copy(idxv.at[0, 0], idxv_v.at[b], sem_i[b]).wait()
        pltpu.make_async_copy(idxu.at[0], idxu_v.at[b], sem_i[b]).wait()

    def fire_gathers(b):
        pltpu.async_copy(tab.at[idxv_v.at[b]], rows_v.at[b], sem_g[b])

    def wait_gathers(b):
        pltpu.make_async_copy(tab.at[idxv_v.at[b]], rows_v.at[b],
                              sem_g[b]).wait()

    def do_scatters(b):
        pltpu.sync_copy(rows_v.at[b], accum.at[idxu_v.at[b]], add=True)

    for p in range(2):
        q = 2 * c + p
        pltpu.sync_copy(zeros, accum.at[pl.ds(s * RPT, RPT), :])
        plsc.subcore_barrier()

        g0 = s * GPT
        pltpu.sync_copy(idxv.at[q, g0], idxv_v.at[0])
        pltpu.sync_copy(idxu.at[g0], idxu_v.at[0])
        fire_gathers(0)
        fire_idx(q, g0 + 1, 1)

        def pair(h, carry, q=q, g0=g0):
            for b in (0, 1):
                gi = 2 * h + b
                bn = 1 - b
                wait_gathers(b)

                @pl.when(gi + 1 < GPT)
                def _(bn=bn):
                    wait_idx(bn)
                    fire_gathers(bn)
                do_scatters(b)

                @pl.when(gi + 2 < GPT)
                def _(q=q, g0=g0, gi=gi, b=b):
                    fire_idx(q, g0 + gi + 2, b)
            return carry

        lax.fori_loop(0, GPT // 2, pair, 0, unroll=False)
        plsc.subcore_barrier()
        pltpu.sync_copy(accum.at[pl.ds(s * RPT, RPT), :],
                        out.at[q, pl.ds(s * RPT, RPT), :])
        plsc.subcore_barrier()


@functools.cache
def _get_sc_call():
    mesh = plsc.VectorSubcoreMesh(
        core_axis_name="c", subcore_axis_name="s", num_cores=2, num_subcores=16)
    return pl.kernel(
        _sc_scatter,
        out_type=jax.ShapeDtypeStruct((4, ACC_ROWS, 32), jnp.bfloat16),
        mesh=mesh,
        scratch_types=[
            pltpu.VMEM((2, GRP), jnp.int32),
            pltpu.VMEM((2, GRP), jnp.int32),
            pltpu.VMEM((2, GRP, 32), jnp.bfloat16),
            pltpu.VMEM_SHARED((ACC_ROWS, 32), jnp.bfloat16),
            pltpu.SemaphoreType.DMA,
            pltpu.SemaphoreType.DMA,
            pltpu.SemaphoreType.DMA,
            pltpu.SemaphoreType.DMA,
        ],
        compiler_params=pltpu.CompilerParams(use_tc_tiling_on_sc=False),
    )


def _row_spec(nc):
    return pl.BlockSpec((RBLK, nc), lambda r: (r, 0))


def _full_spec(shape):
    nd = len(shape)
    return pl.BlockSpec(shape, lambda r, _n=nd: (0,) * _n)


def kernel(ctrs, feats, edge_u_ps, edge_v_ps, left_u, left_v, right_u, right_v, idcs, W_in1, b_in1, W_in2, g_in, b_in, W_seg1, b_seg1, W_seg2, g_seg, b_seg, W_ctr, W_ps, W_left, W_right, g_norm, b_norm, W_ctr2, g_ctr2, b_ctr2):
    f32 = jnp.float32
    i32 = jnp.int32

    Wcat = jnp.concatenate(
        [W_ctr[:, None], W_ps, W_left[:, None], W_right[:, None]], axis=1)
    Wcat = Wcat.transpose(0, 2, 1, 3).reshape(ITER, D, CAT)

    offs = 4 * jnp.arange(NPS, dtype=i32)[:, None]
    npad = E_PAD - E_TOT
    rv = jnp.concatenate([
        (edge_v_ps.astype(i32) * 56 + offs).reshape(-1),
        left_v.astype(i32) * 56 + 48,
        right_v.astype(i32) * 56 + 52,
        (jnp.arange(npad, dtype=i32) % 64) * 56,
    ])
    idxv4 = (rv[None, :] + jnp.arange(4, dtype=i32)[:, None]).reshape(4, NGRP, GRP)
    idxu = jnp.concatenate([
        edge_u_ps.astype(i32).reshape(-1),
        left_u.astype(i32),
        right_u.astype(i32),
        N + (jnp.arange(npad, dtype=i32) % 48),
    ]).reshape(NGRP, GRP)
    zeros = jnp.zeros((RPT, 32), jnp.bfloat16)

    feat = pl.pallas_call(
        _k1_body,
        grid=(GRID,),
        in_specs=[
            _row_spec(2), _row_spec(2),
            _full_spec((2, D)), _full_spec((1, D)), _full_spec((D, D)),
            _full_spec((1, D)), _full_spec((1, D)),
            _full_spec((2, D)), _full_spec((1, D)), _full_spec((D, D)),
            _full_spec((1, D)), _full_spec((1, D)),
        ],
        out_specs=_row_spec(D),
        out_shape=jax.ShapeDtypeStruct((N, D), f32),
    )(ctrs, feats,
      W_in1, b_in1.reshape(1, D), W_in2, g_in.reshape(1, D), b_in.reshape(1, D),
      W_seg1, b_seg1.reshape(1, D), W_seg2, g_seg.reshape(1, D), b_seg.reshape(1, D))

    k2 = pl.pallas_call(
        _k2_body,
        grid=(GRID,),
        in_specs=[_row_spec(D), _full_spec((D, CAT))],
        out_specs=[_row_spec(D), _row_spec(RCAT)],
        out_shape=[jax.ShapeDtypeStruct((N, D), f32),
                   jax.ShapeDtypeStruct((N, RCAT), jnp.bfloat16)],
    )

    k3 = pl.pallas_call(
        _k3_body,
        grid=(GRID,),
        in_specs=[
            _row_spec(D),
            pl.BlockSpec((4, RBLK, 32), lambda r: (0, r, 0)),
            _row_spec(D),
            _full_spec((1, D)), _full_spec((1, D)), _full_spec((D, D)),
            _full_spec((1, D)), _full_spec((1, D)),
        ],
        out_specs=_row_spec(D),
        out_shape=jax.ShapeDtypeStruct((N, D), f32),
    )

    res = feat
    for i in range(ITER):
        ybase, yrel = k2(feat, Wcat[i])
        tab = yrel.reshape(TAB_ROWS, 32)
        scat = _get_sc_call()(tab, idxv4, idxu, zeros)
        feat = k3(ybase, scat, res,
                  g_norm[i].reshape(1, D), b_norm[i].reshape(1, D),
                  W_ctr2[i],
                  g_ctr2[i].reshape(1, D), b_ctr2[i].reshape(1, D))
        res = feat
    return (feat, idcs, ctrs)


# speedup vs baseline: 1.3089x; 1.3089x over previous
import functools

import jax
import jax.numpy as jnp
from jax import lax
from jax.experimental import pallas as pl
from jax.experimental.pallas import tpu as pltpu
from jax.experimental.pallas import tpu_sc as plsc

N = 50000
D = 128
NPS = 12
ITER = 4
NREL = 14
TAB_ROWS = NREL * N * 4
ELR = 10000
E_TOT = NPS * N + 2 * ELR
GRP = 448
GPT = 88
NGRP = 16 * GPT
E_PAD = NGRP * GRP
RPT = 3128
ACC_ROWS = 16 * RPT
RBLK = 2000
GRID = N // RBLK

_HI = lax.Precision.HIGHEST


def _gn_tile(t, g, b):
    m = jnp.sum(t, axis=1, keepdims=True) * (1.0 / D)
    d = t - m
    v = jnp.sum(d * d, axis=1, keepdims=True) * (1.0 / D)
    return d * lax.rsqrt(v + 1e-5) * g + b


def _k1_body(c_ref, f_ref, w1, b1, w2, g1, bb1, ws1, bs1, ws2, gs, bbs, o_ref):
    def branch(inp, wa, ba, wb, g, b):
        x = inp[...]
        h = x[:, 0:1] * wa[0:1, :] + x[:, 1:2] * wa[1:2, :] + ba[...]
        h = jnp.maximum(h, 0.0)
        h = jnp.dot(h, wb[...], precision=_HI)
        return _gn_tile(h, g[...], b[...])

    x = branch(c_ref, w1, b1, w2, g1, bb1)
    y = branch(f_ref, ws1, bs1, ws2, gs, bbs)
    o_ref[...] = jnp.maximum(x + y, 0.0)


def _k2_body(f_ref, w_ref, o_ref):
    o_ref[...] = jnp.dot(f_ref[...], w_ref[0], precision=_HI)[None]


def _k3_body(wc_ref, s_ref, r_ref, gn_ref, bn_ref, w2_ref, g2_ref, b2_ref, o_ref):
    scat = jnp.concatenate(
        [s_ref[0], s_ref[1], s_ref[2], s_ref[3]], axis=1)
    temp = jnp.dot(r_ref[...], wc_ref[...], precision=_HI) + scat
    h = jnp.maximum(_gn_tile(temp, gn_ref[...], bn_ref[...]), 0.0)
    f2 = _gn_tile(jnp.dot(h, w2_ref[...], precision=_HI), g2_ref[...], b2_ref[...])
    o_ref[...] = jnp.maximum(f2 + r_ref[...], 0.0)


def _sc_scatter(tab, idxv, idxu, zeros, out,
                idxv_v, idxu_v, rows_v, accum,
                sem_g0, sem_g1, sem_i0, sem_i1):
    c = lax.axis_index("c")
    s = lax.axis_index("s")
    sem_g = (sem_g0, sem_g1)
    sem_i = (sem_i0, sem_i1)

    def fire_idx(q, g, b):
        pltpu.async_copy(idxv.at[q, g], idxv_v.at[b], sem_i[b])
        pltpu.async_copy(idxu.at[g], idxu_v.at[b], sem_i[b])

    def wait_idx(b):
        pltpu.make_async_copy(idxv.at[0, 0], idxv_v.at[b], sem_i[b]).wait()
        pltpu.make_async_copy(idxu.at[0], idxu_v.at[b], sem_i[b]).wait()

    def fire_gathers(b):
        pltpu.async_copy(tab.at[idxv_v.at[b]], rows_v.at[b], sem_g[b])

    def wait_gathers(b):
        pltpu.make_async_copy(tab.at[idxv_v.at[b]], rows_v.at[b],
                              sem_g[b]).wait()

    def do_scatters(b):
        pltpu.sync_copy(rows_v.at[b], accum.at[idxu_v.at[b]], add=True)

    for p in range(2):
        q = 2 * c + p
        pltpu.sync_copy(zeros, accum.at[pl.ds(s * RPT, RPT), :])
        plsc.subcore_barrier()

        g0 = s * GPT
        pltpu.sync_copy(idxv.at[q, g0], idxv_v.at[0])
        pltpu.sync_copy(idxu.at[g0], idxu_v.at[0])
        fire_gathers(0)
        fire_idx(q, g0 + 1, 1)

        def pair(h, carry, q=q, g0=g0):
            for b in (0, 1):
                gi = 2 * h + b
                bn = 1 - b
                wait_gathers(b)

                @pl.when(gi + 1 < GPT)
                def _(bn=bn):
                    wait_idx(bn)
                    fire_gathers(bn)
                do_scatters(b)

                @pl.when(gi + 2 < GPT)
                def _(q=q, g0=g0, gi=gi, b=b):
                    fire_idx(q, g0 + gi + 2, b)
            return carry

        lax.fori_loop(0, GPT // 2, pair, 0, unroll=False)
        plsc.subcore_barrier()
        pltpu.sync_copy(accum.at[pl.ds(s * RPT, RPT), :],
                        out.at[q, pl.ds(s * RPT, RPT), :])
        plsc.subcore_barrier()


@functools.cache
def _get_sc_call():
    mesh = plsc.VectorSubcoreMesh(
        core_axis_name="c", subcore_axis_name="s", num_cores=2, num_subcores=16)
    return pl.kernel(
        _sc_scatter,
        out_type=jax.ShapeDtypeStruct((4, ACC_ROWS, 32), jnp.float32),
        mesh=mesh,
        scratch_types=[
            pltpu.VMEM((2, GRP), jnp.int32),
            pltpu.VMEM((2, GRP), jnp.int32),
            pltpu.VMEM((2, GRP, 32), jnp.float32),
            pltpu.VMEM_SHARED((ACC_ROWS, 32), jnp.float32),
            pltpu.SemaphoreType.DMA,
            pltpu.SemaphoreType.DMA,
            pltpu.SemaphoreType.DMA,
            pltpu.SemaphoreType.DMA,
        ],
        compiler_params=pltpu.CompilerParams(use_tc_tiling_on_sc=False),
    )


def _row_spec(nc):
    return pl.BlockSpec((RBLK, nc), lambda r: (r, 0))


def _full_spec(shape):
    nd = len(shape)
    return pl.BlockSpec(shape, lambda r, _n=nd: (0,) * _n)


def kernel(ctrs, feats, edge_u_ps, edge_v_ps, left_u, left_v, right_u, right_v, idcs, W_in1, b_in1, W_in2, g_in, b_in, W_seg1, b_seg1, W_seg2, g_seg, b_seg, W_ctr, W_ps, W_left, W_right, g_norm, b_norm, W_ctr2, g_ctr2, b_ctr2):
    f32 = jnp.float32
    i32 = jnp.int32

    Wrel = jnp.concatenate(
        [W_ps, W_left[:, None], W_right[:, None]], axis=1)

    offs = (N * 4) * jnp.arange(NPS, dtype=i32)[:, None]
    npad = E_PAD - E_TOT
    rv = jnp.concatenate([
        (edge_v_ps.astype(i32) * 4 + offs).reshape(-1),
        left_v.astype(i32) * 4 + (N * 4) * 12,
        right_v.astype(i32) * 4 + (N * 4) * 13,
        (jnp.arange(npad, dtype=i32) % 64) * 4,
    ])
    idxv4 = (rv[None, :] + jnp.arange(4, dtype=i32)[:, None]).reshape(4, NGRP, GRP)
    idxu = jnp.concatenate([
        edge_u_ps.astype(i32).reshape(-1),
        left_u.astype(i32),
        right_u.astype(i32),
        N + (jnp.arange(npad, dtype=i32) % 48),
    ]).reshape(NGRP, GRP)
    zeros = jnp.zeros((RPT, 32), f32)

    feat = pl.pallas_call(
        _k1_body,
        grid=(GRID,),
        in_specs=[
            _row_spec(2), _row_spec(2),
            _full_spec((2, D)), _full_spec((1, D)), _full_spec((D, D)),
            _full_spec((1, D)), _full_spec((1, D)),
            _full_spec((2, D)), _full_spec((1, D)), _full_spec((D, D)),
            _full_spec((1, D)), _full_spec((1, D)),
        ],
        out_specs=_row_spec(D),
        out_shape=jax.ShapeDtypeStruct((N, D), f32),
    )(ctrs, feats,
      W_in1, b_in1.reshape(1, D), W_in2, g_in.reshape(1, D), b_in.reshape(1, D),
      W_seg1, b_seg1.reshape(1, D), W_seg2, g_seg.reshape(1, D), b_seg.reshape(1, D))

    k2 = pl.pallas_call(
        _k2_body,
        grid=(GRID, NREL),
        in_specs=[
            pl.BlockSpec((RBLK, D), lambda r, j: (r, 0)),
            pl.BlockSpec((1, D, D), lambda r, j: (j, 0, 0)),
        ],
        out_specs=pl.BlockSpec((1, RBLK, D), lambda r, j: (j, r, 0)),
        out_shape=jax.ShapeDtypeStruct((NREL, N, D), f32),
    )

    k3 = pl.pallas_call(
        _k3_body,
        grid=(GRID,),
        in_specs=[
            _full_spec((D, D)),
            pl.BlockSpec((4, RBLK, 32), lambda r: (0, r, 0)),
            _row_spec(D),
            _full_spec((1, D)), _full_spec((1, D)), _full_spec((D, D)),
            _full_spec((1, D)), _full_spec((1, D)),
        ],
        out_specs=_row_spec(D),
        out_shape=jax.ShapeDtypeStruct((N, D), f32),
    )

    res = feat
    for i in range(ITER):
        yrel = k2(feat, Wrel[i])
        tab = yrel.reshape(TAB_ROWS, 32)
        scat = _get_sc_call()(tab, idxv4, idxu, zeros)
        feat = k3(W_ctr[i], scat, res,
                  g_norm[i].reshape(1, D), b_norm[i].reshape(1, D),
                  W_ctr2[i],
                  g_ctr2[i].reshape(1, D), b_ctr2[i].reshape(1, D))
        res = feat
    return (feat, idcs, ctrs)
